# baseline (speedup 1.0000x reference)
import jax
import jax.numpy as jnp
from jax.experimental import pallas as pl
from jax.experimental.pallas import tpu as pltpu

EMBED = 2048
EXPERTS = 16
CHUNK = 256
NBUF = 16


def _gating_body(x_hbm, w_ref, b_ref, o_ref, buf, sem):
    i = pl.program_id(0)
    nsteps = pl.num_programs(0)
    rows = nsteps * CHUNK
    x2 = x_hbm.reshape(rows, EMBED)

    def _copy(step, slot):
        return pltpu.make_async_copy(
            x2.at[pl.ds(step * CHUNK, CHUNK), :],
            buf.at[slot],
            sem.at[slot],
        )

    @pl.when(i == 0)
    def _():
        for k in range(NBUF - 1):
            _copy(k, k).start()

    nxt = i + NBUF - 1

    @pl.when(nxt < nsteps)
    def _():
        _copy(nxt, jax.lax.rem(nxt, NBUF)).start()

    slot = jax.lax.rem(i, NBUF)
    _copy(i, slot).wait()

    xb = buf[slot]
    scores = jax.lax.dot_general(
        xb, w_ref[...], (((1,), (1,)), ((), ())), preferred_element_type=jnp.float32
    )
    scores = scores + b_ref[...]
    m = jnp.max(scores, axis=1, keepdims=True)
    e = jnp.exp(scores - m)
    p = e / jnp.sum(e, axis=1, keepdims=True)
    o_ref[...] = p.reshape(o_ref.shape)


def kernel(x, W, b):
    target_length, batch_size, embed_dim = x.shape
    rows = target_length * batch_size
    b2 = b.reshape(1, EXPERTS)
    nsteps = rows // CHUNK
    t_tile = CHUNK // batch_size
    return pl.pallas_call(
        _gating_body,
        grid=(nsteps,),
        in_specs=[
            pl.BlockSpec(memory_space=pl.ANY),
            pl.BlockSpec((EXPERTS, embed_dim), lambda i: (0, 0)),
            pl.BlockSpec((1, EXPERTS), lambda i: (0, 0)),
        ],
        out_specs=pl.BlockSpec((t_tile, batch_size, EXPERTS), lambda i: (i, 0, 0)),
        out_shape=jax.ShapeDtypeStruct(
            (target_length, batch_size, EXPERTS), jnp.float32
        ),
        scratch_shapes=[
            pltpu.VMEM((NBUF, CHUNK, EMBED), jnp.float32),
            pltpu.SemaphoreType.DMA((NBUF,)),
        ],
    )(x, W, b2)


# speedup vs baseline: 1.0020x; 1.0020x over previous
import jax
import jax.numpy as jnp
from jax.experimental import pallas as pl
from jax.experimental.pallas import tpu as pltpu

EMBED = 2048
EXPERTS = 16
CHUNK = 512
NBUF = 12


def _gating_body(x_hbm, w_ref, b_ref, o_ref, buf, sem):
    i = pl.program_id(0)
    nsteps = pl.num_programs(0)
    rows = nsteps * CHUNK
    x2 = x_hbm.reshape(rows, EMBED)

    def _copy(step, slot):
        return pltpu.make_async_copy(
            x2.at[pl.ds(step * CHUNK, CHUNK), :],
            buf.at[slot],
            sem.at[slot],
        )

    @pl.when(i == 0)
    def _():
        for k in range(NBUF - 1):
            _copy(k, k).start()

    nxt = i + NBUF - 1

    @pl.when(nxt < nsteps)
    def _():
        _copy(nxt, jax.lax.rem(nxt, NBUF)).start()

    slot = jax.lax.rem(i, NBUF)
    _copy(i, slot).wait()

    xb = buf[slot]
    scores = jax.lax.dot_general(
        xb, w_ref[...], (((1,), (1,)), ((), ())), preferred_element_type=jnp.float32
    )
    scores = scores + b_ref[...]
    m = jnp.max(scores, axis=1, keepdims=True)
    e = jnp.exp(scores - m)
    p = e / jnp.sum(e, axis=1, keepdims=True)
    o_ref[...] = p.reshape(o_ref.shape)


def kernel(x, W, b):
    target_length, batch_size, embed_dim = x.shape
    rows = target_length * batch_size
    b2 = b.reshape(1, EXPERTS)
    nsteps = rows // CHUNK
    t_tile = CHUNK // batch_size
    return pl.pallas_call(
        _gating_body,
        grid=(nsteps,),
        in_specs=[
            pl.BlockSpec(memory_space=pl.ANY),
            pl.BlockSpec((EXPERTS, embed_dim), lambda i: (0, 0)),
            pl.BlockSpec((1, EXPERTS), lambda i: (0, 0)),
        ],
        out_specs=pl.BlockSpec((t_tile, batch_size, EXPERTS), lambda i: (i, 0, 0)),
        out_shape=jax.ShapeDtypeStruct(
            (target_length, batch_size, EXPERTS), jnp.float32
        ),
        scratch_shapes=[
            pltpu.VMEM((NBUF, CHUNK, EMBED), jnp.float32),
            pltpu.SemaphoreType.DMA((NBUF,)),
        ],
    )(x, W, b2)
